# baseline (speedup 1.0000x reference)
import functools

import jax
import jax.numpy as jnp
from jax import lax
from jax.experimental import pallas as pl
from jax.experimental.pallas import tpu as pltpu
from jax.experimental.pallas import tpu_sc as plsc

SZ = 2048
TEMP = 10.0
EPS = 1e-20
LN2 = 0.6931471805599453

SC_ROWS = 2048
TC_ROWS_PER_BLOCK = 64

_NC, _NS = 2, 16
_NW = _NC * _NS


def _native_view(x):
    return (
        x.reshape(SZ, 16, 128, 2)
        .transpose(0, 1, 3, 2)
        .reshape(SZ, 32, 128)
    )


def _tc_body(g_ref, u_ref, o_ref):
    for g in range(16):
        a0 = g_ref[:, 2 * g, :]
        a1 = g_ref[:, 2 * g + 1, :]
        u0 = u_ref[:, 2 * g, :]
        u1 = u_ref[:, 2 * g + 1, :]
        L0 = EPS - jnp.log(u0 + EPS)
        L1 = EPS - jnp.log(u1 + EPS)
        lr = jnp.log(L0 / L1)
        s = (a0 - a1 - lr) * (1.0 / TEMP)
        o_ref[:, 128 * g:128 * (g + 1)] = 1.0 / (1.0 + jnp.exp(-s))


def _tc_call(gm, uu, row0, nrows):
    grid = nrows // TC_ROWS_PER_BLOCK
    off = row0 // TC_ROWS_PER_BLOCK
    return pl.pallas_call(
        _tc_body,
        grid=(grid,),
        in_specs=[
            pl.BlockSpec((TC_ROWS_PER_BLOCK, 32, 128), lambda i: (i + off, 0, 0)),
            pl.BlockSpec((TC_ROWS_PER_BLOCK, 32, 128), lambda i: (i + off, 0, 0)),
        ],
        out_specs=pl.BlockSpec((TC_ROWS_PER_BLOCK, SZ), lambda i: (i, 0)),
        out_shape=jax.ShapeDtypeStruct((nrows, SZ), jnp.float32),
    )(gm, uu)


def _ln(v):
    b = lax.bitcast_convert_type(v, jnp.int32)
    e = lax.shift_right_arithmetic(b, 23) - 127
    m = lax.bitcast_convert_type(
        (b & 0x007FFFFF) | 0x3F800000, jnp.float32)
    big = m > 1.4142135
    m = jnp.where(big, m * 0.5, m)
    e = jnp.where(big, e + 1, e)
    ef = e.astype(jnp.float32)
    r = (m - 1.0) / (m + 1.0)
    r2 = r * r
    p = 1.0 / 9.0
    p = 1.0 / 7.0 + r2 * p
    p = 1.0 / 5.0 + r2 * p
    p = 1.0 / 3.0 + r2 * p
    p = 1.0 + r2 * p
    return ef * LN2 + 2.0 * r * p


def _sc_group(gbuf, ubuf, obuf, g, j):
    a0 = gbuf[2 * g, pl.ds(16 * j, 16)]
    a1 = gbuf[2 * g + 1, pl.ds(16 * j, 16)]
    u0 = ubuf[2 * g, pl.ds(16 * j, 16)]
    u1 = ubuf[2 * g + 1, pl.ds(16 * j, 16)]
    L0 = EPS - _ln(u0 + EPS)
    L1 = EPS - _ln(u1 + EPS)
    lr = _ln(L0 / L1)
    s = (a1 - a0 + lr) * (1.0 / TEMP)
    obuf[pl.ds(128 * g + 16 * j, 16)] = 1.0 / (1.0 + jnp.exp(s))


def _sc_body(nrows, g_hbm, u_hbm, o_hbm, gbuf, ubuf, obuf):
    wid = lax.axis_index("s") * _NC + lax.axis_index("c")
    rpw = nrows // _NW
    base = wid * rpw

    def row_step(t, carry):
        row = base + t
        pltpu.sync_copy(g_hbm.at[row], gbuf)
        pltpu.sync_copy(u_hbm.at[row], ubuf)

        def col_step(g, carry2):
            for j in range(8):
                _sc_group(gbuf, ubuf, obuf, g, j)
            return carry2

        lax.fori_loop(0, 16, col_step, 0)
        pltpu.sync_copy(obuf, o_hbm.at[row])
        return carry

    lax.fori_loop(0, rpw, row_step, 0)


def _sc_call(gm, uu, nrows):
    body = functools.partial(_sc_body, nrows)
    mesh = plsc.VectorSubcoreMesh(core_axis_name="c", subcore_axis_name="s")
    fn = pl.kernel(
        body,
        out_type=jax.ShapeDtypeStruct((nrows, SZ), jnp.float32),
        mesh=mesh,
        scratch_types=[
            pltpu.VMEM((32, 128), jnp.float32),
            pltpu.VMEM((32, 128), jnp.float32),
            pltpu.VMEM((SZ,), jnp.float32),
        ],
    )
    return fn(gm, uu)


def kernel(gen_matrix, u):
    gm = _native_view(gen_matrix.reshape(SZ, SZ, 2))
    uu = _native_view(u.reshape(SZ, SZ, 2))
    if SC_ROWS == 0:
        return _tc_call(gm, uu, 0, SZ)
    if SC_ROWS == SZ:
        return _sc_call(gm, uu, SZ)
    top = _sc_call(gm, uu, SC_ROWS)
    bot = _tc_call(gm, uu, SC_ROWS, SZ - SC_ROWS)
    return jnp.concatenate([top, bot], axis=0)


# speedup vs baseline: 5.9069x; 5.9069x over previous
import functools

import jax
import jax.numpy as jnp
from jax import lax
from jax.experimental import pallas as pl
from jax.experimental.pallas import tpu as pltpu
from jax.experimental.pallas import tpu_sc as plsc

SZ = 2048
TEMP = 10.0
EPS = 1e-20
LN2 = 0.6931471805599453

SC_ROWS = 256
TC_ROWS_PER_BLOCK = 64

_NC, _NS = 2, 16
_NW = _NC * _NS


def _native_view(x):
    return (
        x.reshape(SZ, 16, 128, 2)
        .transpose(0, 1, 3, 2)
        .reshape(SZ, 32, 128)
    )


def _tc_body(g_ref, u_ref, o_ref):
    for g in range(16):
        a0 = g_ref[:, 2 * g, :]
        a1 = g_ref[:, 2 * g + 1, :]
        u0 = u_ref[:, 2 * g, :]
        u1 = u_ref[:, 2 * g + 1, :]
        L0 = EPS - jnp.log(u0 + EPS)
        L1 = EPS - jnp.log(u1 + EPS)
        lr = jnp.log(L0 / L1)
        s = (a0 - a1 - lr) * (1.0 / TEMP)
        o_ref[:, 128 * g:128 * (g + 1)] = 1.0 / (1.0 + jnp.exp(-s))


def _tc_call(gm, uu, row0, nrows):
    grid = nrows // TC_ROWS_PER_BLOCK
    off = row0 // TC_ROWS_PER_BLOCK
    return pl.pallas_call(
        _tc_body,
        grid=(grid,),
        in_specs=[
            pl.BlockSpec((TC_ROWS_PER_BLOCK, 32, 128), lambda i: (i + off, 0, 0)),
            pl.BlockSpec((TC_ROWS_PER_BLOCK, 32, 128), lambda i: (i + off, 0, 0)),
        ],
        out_specs=pl.BlockSpec((TC_ROWS_PER_BLOCK, SZ), lambda i: (i, 0)),
        out_shape=jax.ShapeDtypeStruct((nrows, SZ), jnp.float32),
    )(gm, uu)


def _ln(v):
    b = lax.bitcast_convert_type(v, jnp.int32)
    e = lax.shift_right_arithmetic(b, 23) - 127
    m = lax.bitcast_convert_type(
        (b & 0x007FFFFF) | 0x3F800000, jnp.float32)
    big = m > 1.4142135
    m = jnp.where(big, m * 0.5, m)
    e = jnp.where(big, e + 1, e)
    ef = e.astype(jnp.float32)
    r = (m - 1.0) / (m + 1.0)
    r2 = r * r
    p = 1.0 / 9.0
    p = 1.0 / 7.0 + r2 * p
    p = 1.0 / 5.0 + r2 * p
    p = 1.0 / 3.0 + r2 * p
    p = 1.0 + r2 * p
    return ef * LN2 + 2.0 * r * p


def _sc_group(gbuf, ubuf, obuf, g, j):
    a0 = gbuf[2 * g, pl.ds(16 * j, 16)]
    a1 = gbuf[2 * g + 1, pl.ds(16 * j, 16)]
    u0 = ubuf[2 * g, pl.ds(16 * j, 16)]
    u1 = ubuf[2 * g + 1, pl.ds(16 * j, 16)]
    L0 = EPS - _ln(u0 + EPS)
    L1 = EPS - _ln(u1 + EPS)
    lr = _ln(L0 / L1)
    s = (a1 - a0 + lr) * (1.0 / TEMP)
    obuf[pl.ds(128 * g + 16 * j, 16)] = 1.0 / (1.0 + jnp.exp(s))


def _sc_body(nrows, g_hbm, u_hbm, o_hbm, gbuf, ubuf, obuf):
    wid = lax.axis_index("s") * _NC + lax.axis_index("c")
    rpw = nrows // _NW
    base = wid * rpw

    def row_step(t, carry):
        row = base + t
        pltpu.sync_copy(g_hbm.at[row], gbuf)
        pltpu.sync_copy(u_hbm.at[row], ubuf)

        def col_step(g, carry2):
            for j in range(8):
                _sc_group(gbuf, ubuf, obuf, g, j)
            return carry2

        lax.fori_loop(0, 16, col_step, 0)
        pltpu.sync_copy(obuf, o_hbm.at[row])
        return carry

    lax.fori_loop(0, rpw, row_step, 0)


def _sc_call(gm, uu, nrows):
    body = functools.partial(_sc_body, nrows)
    mesh = plsc.VectorSubcoreMesh(core_axis_name="c", subcore_axis_name="s")
    fn = pl.kernel(
        body,
        out_type=jax.ShapeDtypeStruct((nrows, SZ), jnp.float32),
        mesh=mesh,
        scratch_types=[
            pltpu.VMEM((32, 128), jnp.float32),
            pltpu.VMEM((32, 128), jnp.float32),
            pltpu.VMEM((SZ,), jnp.float32),
        ],
    )
    return fn(gm, uu)


def kernel(gen_matrix, u):
    gm = _native_view(gen_matrix.reshape(SZ, SZ, 2))
    uu = _native_view(u.reshape(SZ, SZ, 2))
    if SC_ROWS == 0:
        return _tc_call(gm, uu, 0, SZ)
    if SC_ROWS == SZ:
        return _sc_call(gm, uu, SZ)
    top = _sc_call(gm, uu, SC_ROWS)
    bot = _tc_call(gm, uu, SC_ROWS, SZ - SC_ROWS)
    return jnp.concatenate([top, bot], axis=0)
